# baseline (speedup 1.0000x reference)
import functools

import jax
import jax.numpy as jnp
from jax import lax
from jax.experimental import pallas as pl
from jax.experimental.pallas import tpu as pltpu
from jax.experimental.pallas import tpu_sc as plsc

B = 16384
N_OBJ = 64
N_ACT = 8
P_SPACE = 0.1
EXIST_THR = 0.8

B_SC = 1024
B_TC = B - B_SC

NC, NS, L = 2, 16, 16
NW = NC * NS
ROWS_PER_W = B_SC // NW
NB = 16
TILES = ROWS_PER_W // NB
GROW = 256
GAT = NB * GROW


TB = 512


def _tc_body(x_blk, act_v, mask_v, ap, pv):
    xb = x_blk[...]
    io1 = lax.broadcasted_iota(jnp.int32, (1, N_OBJ, N_OBJ), 1)
    io2 = lax.broadcasted_iota(jnp.int32, (1, N_OBJ, N_OBJ), 2)
    eye = io1 == io2
    d = jnp.sum(jnp.where(eye, xb, 0.0), axis=2)
    eq = (d > EXIST_THR) == (mask_v[...] > 0)
    eqall = jnp.all(eq, axis=1, keepdims=True)
    a_col = x_blk[:, 0, 0:1]
    b_col = x_blk[:, 1, 0:1]
    p = jnp.abs(a_col - b_col)
    satf = jnp.where(eqall & (p < P_SPACE), 1.0, 0.0)
    pv[...] = p
    an = act_v[...] / (act_v[...] + 1e-20)
    ap[...] = satf * an


def _tc_run(x, act_row, mask_row):
    return pl.pallas_call(
        _tc_body,
        grid=(B_TC // TB,),
        out_shape=[
            jax.ShapeDtypeStruct((B_TC, N_ACT), jnp.float32),
            jax.ShapeDtypeStruct((B_TC, 1), jnp.float32),
        ],
        in_specs=[
            pl.BlockSpec((TB, N_OBJ, N_OBJ), lambda t: (t, 0, 0)),
            pl.BlockSpec((1, N_ACT), lambda t: (0, 0)),
            pl.BlockSpec((1, N_OBJ), lambda t: (0, 0)),
        ],
        out_specs=[
            pl.BlockSpec((TB, N_ACT), lambda t: (t, 0)),
            pl.BlockSpec((TB, 1), lambda t: (t, 0)),
        ],
    )(x, act_row, mask_row)


def _sc_body(x_hbm, act_hbm, mask_hbm, ap_hbm, pv_hbm,
             gat_v, mask_v, mexp_v, act_v, sat_v, ap_v, pv_v, sem):
    wid = lax.axis_index("s") * NC + lax.axis_index("c")
    base_row = B_TC + wid * ROWS_PER_W

    pltpu.sync_copy(mask_hbm, mask_v)
    pltpu.sync_copy(act_hbm, act_v)

    iota = lax.iota(jnp.int32, L)
    r256 = iota * GROW

    for m in range(N_OBJ // L):
        chunk = mask_v[pl.ds(m * L, L)]
        for j in range(L):
            mexp_v[pl.ds((m * L + j) * L, L)] = jnp.broadcast_to(
                chunk[j], (L,))

    act = act_v[...]
    an = act / (act + 1e-20)
    half = (iota >= 8).astype(jnp.int32)
    zero = jnp.zeros((L,), jnp.int32)

    def tile(t, carry):
        b0 = base_row + t * NB
        pltpu.sync_copy(x_hbm.at[pl.ds(b0 * GROW, GAT)], gat_v)

        a_val = plsc.load_gather(gat_v, [r256, zero])
        b_val = plsc.load_gather(gat_v, [r256 + 4, zero])
        p = jnp.abs(a_val - b_val)
        acc = p < P_SPACE
        for i in range(N_OBJ):
            rows = r256 + (65 * i) // 16
            col = jnp.full((L,), i % 16, jnp.int32)
            diag = plsc.load_gather(gat_v, [rows, col])
            m_i = mexp_v[pl.ds(i * L, L)] > 0
            acc = acc & ((diag > EXIST_THR) == m_i)
        satf = jnp.where(acc, 1.0, 0.0).astype(jnp.float32)

        pv_v[pl.ds(t * NB, NB)] = p
        sat_v[...] = satf
        for pair in range(NB // 2):
            sel = half + 2 * pair
            ap_v[pl.ds(t * NB * N_ACT + pair * L, L)] = (
                plsc.load_gather(sat_v, [sel]) * an)
        return carry

    lax.fori_loop(0, TILES, tile, 0, unroll=False)

    pltpu.sync_copy(pv_v, pv_hbm.at[pl.ds(wid * ROWS_PER_W, ROWS_PER_W)])
    pltpu.sync_copy(ap_v, ap_hbm.at[pl.ds(wid * ROWS_PER_W * N_ACT,
                                          ROWS_PER_W * N_ACT)])


def _sc_run(x2, act2, mask_i32):
    mesh = plsc.VectorSubcoreMesh(core_axis_name="c", subcore_axis_name="s")
    f = functools.partial(
        pl.kernel,
        mesh=mesh,
        compiler_params=pltpu.CompilerParams(needs_layout_passes=False,
                                             use_tc_tiling_on_sc=False),
        out_type=[
            jax.ShapeDtypeStruct((B_SC * N_ACT,), jnp.float32),
            jax.ShapeDtypeStruct((B_SC,), jnp.float32),
        ],
        scratch_types=[
            pltpu.VMEM((GAT, L), jnp.float32),
            pltpu.VMEM((N_OBJ,), jnp.int32),
            pltpu.VMEM((N_OBJ * L,), jnp.int32),
            pltpu.VMEM((L,), jnp.float32),
            pltpu.VMEM((L,), jnp.float32),
            pltpu.VMEM((ROWS_PER_W * N_ACT,), jnp.float32),
            pltpu.VMEM((ROWS_PER_W,), jnp.float32),
            pltpu.SemaphoreType.DMA,
        ],
    )(_sc_body)
    return f(x2, act2, mask_i32)


def kernel(x, action, mask):
    act2 = jnp.concatenate([action, action]).astype(jnp.float32)
    mask_i32 = mask.astype(jnp.int32)
    act_row = action.astype(jnp.float32).reshape(1, N_ACT)
    mask_row = mask.astype(jnp.int32).reshape(1, N_OBJ)

    @jax.jit
    def run(x, act2, mask_i32, act_row, mask_row):
        x2 = x.reshape(B * 256, 16)
        ap_sc, pv_sc = _sc_run(x2, act2, mask_i32)
        ap_tc, pv_tc = _tc_run(x, act_row, mask_row)
        ap = jnp.concatenate([ap_tc, ap_sc.reshape(B_SC, N_ACT)], axis=0)
        pv = jnp.concatenate([pv_tc.reshape(B_TC), pv_sc])
        return ap, pv

    return run(x, act2, mask_i32, act_row, mask_row)


# speedup vs baseline: 1.4558x; 1.4558x over previous
import functools

import jax
import jax.numpy as jnp
from jax import lax
from jax.experimental import pallas as pl
from jax.experimental.pallas import tpu as pltpu
from jax.experimental.pallas import tpu_sc as plsc

B = 16384
N_OBJ = 64
N_ACT = 8
P_SPACE = 0.1
EXIST_THR = 0.8

B_SC = 1024
B_TC = B - B_SC

NC, NS, L = 2, 16, 16
NW = NC * NS
ROWS_PER_W = B_SC // NW
NB = 16
TILES = ROWS_PER_W // NB
GROW = 256
GAT = NB * GROW


TB = 512
ROW = N_OBJ * N_OBJ


def _tc_body(x_blk, act_v, mask_v, eye_v, ap, pv):
    xb = x_blk[...]
    eqf = ((xb > EXIST_THR) == (mask_v[...] > 0)).astype(jnp.float32)
    s = lax.dot_general(eqf, eye_v[...], (((1,), (0,)), ((), ())),
                        preferred_element_type=jnp.float32)
    eqall = s > (N_OBJ - 0.5)
    a_col = xb[:, 0:1]
    b_col = xb[:, N_OBJ:N_OBJ + 1]
    p = jnp.abs(a_col - b_col)
    satf = jnp.where(eqall & (p < P_SPACE), 1.0, 0.0)
    pv[...] = p
    an = act_v[...] / (act_v[...] + 1e-20)
    ap[...] = satf * an


def _tc_run(x4, act_row, mask_4k, eye_col):
    return pl.pallas_call(
        _tc_body,
        grid=(B_TC // TB,),
        out_shape=[
            jax.ShapeDtypeStruct((B_TC, N_ACT), jnp.float32),
            jax.ShapeDtypeStruct((B_TC, 1), jnp.float32),
        ],
        in_specs=[
            pl.BlockSpec((TB, ROW), lambda t: (t, 0)),
            pl.BlockSpec((1, N_ACT), lambda t: (0, 0)),
            pl.BlockSpec((1, ROW), lambda t: (0, 0)),
            pl.BlockSpec((ROW, 1), lambda t: (0, 0)),
        ],
        out_specs=[
            pl.BlockSpec((TB, N_ACT), lambda t: (t, 0)),
            pl.BlockSpec((TB, 1), lambda t: (t, 0)),
        ],
    )(x4, act_row, mask_4k, eye_col)


def _sc_body(x_hbm, act_hbm, mask_hbm, ap_hbm, pv_hbm,
             gat_v, mask_v, mexp_v, act_v, sat_v, ap_v, pv_v, sem):
    wid = lax.axis_index("s") * NC + lax.axis_index("c")
    base_row = B_TC + wid * ROWS_PER_W

    pltpu.sync_copy(mask_hbm, mask_v)
    pltpu.sync_copy(act_hbm, act_v)

    iota = lax.iota(jnp.int32, L)
    r256 = iota * GROW

    for m in range(N_OBJ // L):
        chunk = mask_v[pl.ds(m * L, L)]
        for j in range(L):
            mexp_v[pl.ds((m * L + j) * L, L)] = jnp.broadcast_to(
                chunk[j], (L,))

    act = act_v[...]
    an = act / (act + 1e-20)
    half = (iota >= 8).astype(jnp.int32)
    zero = jnp.zeros((L,), jnp.int32)

    def tile(t, carry):
        b0 = base_row + t * NB
        pltpu.sync_copy(x_hbm.at[pl.ds(b0 * GROW, GAT)], gat_v)

        a_val = plsc.load_gather(gat_v, [r256, zero])
        b_val = plsc.load_gather(gat_v, [r256 + 4, zero])
        p = jnp.abs(a_val - b_val)
        acc = p < P_SPACE
        for i in range(N_OBJ):
            rows = r256 + (65 * i) // 16
            col = jnp.full((L,), i % 16, jnp.int32)
            diag = plsc.load_gather(gat_v, [rows, col])
            m_i = mexp_v[pl.ds(i * L, L)] > 0
            acc = acc & ((diag > EXIST_THR) == m_i)
        satf = jnp.where(acc, 1.0, 0.0).astype(jnp.float32)

        pv_v[pl.ds(t * NB, NB)] = p
        sat_v[...] = satf
        for pair in range(NB // 2):
            sel = half + 2 * pair
            ap_v[pl.ds(t * NB * N_ACT + pair * L, L)] = (
                plsc.load_gather(sat_v, [sel]) * an)
        return carry

    lax.fori_loop(0, TILES, tile, 0, unroll=False)

    pltpu.sync_copy(pv_v, pv_hbm.at[pl.ds(wid * ROWS_PER_W, ROWS_PER_W)])
    pltpu.sync_copy(ap_v, ap_hbm.at[pl.ds(wid * ROWS_PER_W * N_ACT,
                                          ROWS_PER_W * N_ACT)])


def _sc_run(x2, act2, mask_i32):
    mesh = plsc.VectorSubcoreMesh(core_axis_name="c", subcore_axis_name="s")
    f = functools.partial(
        pl.kernel,
        mesh=mesh,
        compiler_params=pltpu.CompilerParams(needs_layout_passes=False,
                                             use_tc_tiling_on_sc=False),
        out_type=[
            jax.ShapeDtypeStruct((B_SC * N_ACT,), jnp.float32),
            jax.ShapeDtypeStruct((B_SC,), jnp.float32),
        ],
        scratch_types=[
            pltpu.VMEM((GAT, L), jnp.float32),
            pltpu.VMEM((N_OBJ,), jnp.int32),
            pltpu.VMEM((N_OBJ * L,), jnp.int32),
            pltpu.VMEM((L,), jnp.float32),
            pltpu.VMEM((L,), jnp.float32),
            pltpu.VMEM((ROWS_PER_W * N_ACT,), jnp.float32),
            pltpu.VMEM((ROWS_PER_W,), jnp.float32),
            pltpu.SemaphoreType.DMA,
        ],
    )(_sc_body)
    return f(x2, act2, mask_i32)


def kernel(x, action, mask):
    act2 = jnp.concatenate([action, action]).astype(jnp.float32)
    mask_i32 = mask.astype(jnp.int32)
    act_row = action.astype(jnp.float32).reshape(1, N_ACT)
    mask_4k = jnp.repeat(mask.astype(jnp.int32), N_OBJ).reshape(1, ROW)
    eye_col = jnp.eye(N_OBJ, dtype=jnp.float32).reshape(ROW, 1)

    @jax.jit
    def run(x, act2, mask_i32, act_row, mask_4k, eye_col):
        x2 = x.reshape(B * 256, 16)
        ap_sc, pv_sc = _sc_run(x2, act2, mask_i32)
        ap_tc, pv_tc = _tc_run(x.reshape(B, ROW), act_row, mask_4k,
                               eye_col)
        ap = jnp.concatenate([ap_tc, ap_sc.reshape(B_SC, N_ACT)], axis=0)
        pv = jnp.concatenate([pv_tc.reshape(B_TC), pv_sc])
        return ap, pv

    return run(x, act2, mask_i32, act_row, mask_4k, eye_col)


# speedup vs baseline: 1.9571x; 1.3443x over previous
import functools

import jax
import jax.numpy as jnp
from jax import lax
from jax.experimental import pallas as pl
from jax.experimental.pallas import tpu as pltpu
from jax.experimental.pallas import tpu_sc as plsc

B = 16384
N_OBJ = 64
N_ACT = 8
P_SPACE = 0.1
EXIST_THR = 0.8

B_SC = 1024
B_TC = B - B_SC

NC, NS, L = 2, 16, 16
NW = NC * NS
ROWS_PER_W = B_SC // NW
NB = 16
TILES = ROWS_PER_W // NB
GROW = 256
GAT = NB * GROW


TB = 512
ROW = N_OBJ * N_OBJ


def _tc_body(x_blk, act_v, mask_v, eye_v, ap, pv):
    xb = x_blk[...]
    eqf = ((xb > EXIST_THR) == (mask_v[...] > 0)).astype(jnp.float32)
    s1 = jnp.sum(eqf * eye_v[...], axis=2)
    s = jnp.sum(s1, axis=1, keepdims=True)
    eqall = s > (N_OBJ - 0.5)
    a_col = x_blk[:, 0, 0:1]
    b_col = x_blk[:, 1, 0:1]
    p = jnp.abs(a_col - b_col)
    satf = jnp.where(eqall & (p < P_SPACE), 1.0, 0.0)
    pv[...] = p
    an = act_v[...] / (act_v[...] + 1e-20)
    ap[...] = satf * an


def _tc_run(x, act_row, mask_3d, eye_3d):
    return pl.pallas_call(
        _tc_body,
        grid=(B_TC // TB,),
        out_shape=[
            jax.ShapeDtypeStruct((B_TC, N_ACT), jnp.float32),
            jax.ShapeDtypeStruct((B_TC, 1), jnp.float32),
        ],
        in_specs=[
            pl.BlockSpec((TB, N_OBJ, N_OBJ), lambda t: (t, 0, 0)),
            pl.BlockSpec((1, N_ACT), lambda t: (0, 0)),
            pl.BlockSpec((1, N_OBJ, N_OBJ), lambda t: (0, 0, 0)),
            pl.BlockSpec((1, N_OBJ, N_OBJ), lambda t: (0, 0, 0)),
        ],
        out_specs=[
            pl.BlockSpec((TB, N_ACT), lambda t: (t, 0)),
            pl.BlockSpec((TB, 1), lambda t: (t, 0)),
        ],
    )(x, act_row, mask_3d, eye_3d)


def _sc_body(x_hbm, act_hbm, mask_hbm, ap_hbm, pv_hbm,
             gat_v, mask_v, mexp_v, act_v, sat_v, ap_v, pv_v, sem):
    wid = lax.axis_index("s") * NC + lax.axis_index("c")
    base_row = wid * ROWS_PER_W

    pltpu.sync_copy(mask_hbm, mask_v)
    pltpu.sync_copy(act_hbm, act_v)

    iota = lax.iota(jnp.int32, L)
    r256 = iota * GROW

    for m in range(N_OBJ // L):
        chunk = mask_v[pl.ds(m * L, L)]
        for j in range(L):
            mexp_v[pl.ds((m * L + j) * L, L)] = jnp.broadcast_to(
                chunk[j], (L,))

    act = act_v[...]
    an = act / (act + 1e-20)
    half = (iota >= 8).astype(jnp.int32)
    zero = jnp.zeros((L,), jnp.int32)

    def tile(t, carry):
        b0 = base_row + t * NB
        pltpu.sync_copy(x_hbm.at[pl.ds(b0 * GROW, GAT)], gat_v)

        a_val = plsc.load_gather(gat_v, [r256, zero])
        b_val = plsc.load_gather(gat_v, [r256 + 4, zero])
        p = jnp.abs(a_val - b_val)
        acc = p < P_SPACE
        for i in range(N_OBJ):
            rows = r256 + (65 * i) // 16
            col = jnp.full((L,), i % 16, jnp.int32)
            diag = plsc.load_gather(gat_v, [rows, col])
            m_i = mexp_v[pl.ds(i * L, L)] > 0
            acc = acc & ((diag > EXIST_THR) == m_i)
        satf = jnp.where(acc, 1.0, 0.0).astype(jnp.float32)

        pv_v[pl.ds(t * NB, NB)] = p
        sat_v[...] = satf
        for pair in range(NB // 2):
            sel = half + 2 * pair
            ap_v[pl.ds(t * NB * N_ACT + pair * L, L)] = (
                plsc.load_gather(sat_v, [sel]) * an)
        return carry

    lax.fori_loop(0, TILES, tile, 0, unroll=False)

    pltpu.sync_copy(pv_v, pv_hbm.at[pl.ds(wid * ROWS_PER_W, ROWS_PER_W)])
    pltpu.sync_copy(ap_v, ap_hbm.at[pl.ds(wid * ROWS_PER_W * N_ACT,
                                          ROWS_PER_W * N_ACT)])


def _sc_run(x2, act2, mask_i32):
    mesh = plsc.VectorSubcoreMesh(core_axis_name="c", subcore_axis_name="s")
    f = functools.partial(
        pl.kernel,
        mesh=mesh,
        compiler_params=pltpu.CompilerParams(needs_layout_passes=False,
                                             use_tc_tiling_on_sc=False),
        out_type=[
            jax.ShapeDtypeStruct((B_SC * N_ACT,), jnp.float32),
            jax.ShapeDtypeStruct((B_SC,), jnp.float32),
        ],
        scratch_types=[
            pltpu.VMEM((GAT, L), jnp.float32),
            pltpu.VMEM((N_OBJ,), jnp.int32),
            pltpu.VMEM((N_OBJ * L,), jnp.int32),
            pltpu.VMEM((L,), jnp.float32),
            pltpu.VMEM((L,), jnp.float32),
            pltpu.VMEM((ROWS_PER_W * N_ACT,), jnp.float32),
            pltpu.VMEM((ROWS_PER_W,), jnp.float32),
            pltpu.SemaphoreType.DMA,
        ],
    )(_sc_body)
    return f(x2, act2, mask_i32)


def kernel(x, action, mask):
    act2 = jnp.concatenate([action, action]).astype(jnp.float32)
    mask_i32 = mask.astype(jnp.int32)
    act_row = action.astype(jnp.float32).reshape(1, N_ACT)
    mask_3d = jnp.broadcast_to(
        mask.astype(jnp.int32)[None, :, None], (1, N_OBJ, N_OBJ))
    eye_3d = jnp.eye(N_OBJ, dtype=jnp.float32).reshape(1, N_OBJ, N_OBJ)

    @jax.jit
    def run(x, act2, mask_i32, act_row, mask_3d, eye_3d):
        x2 = lax.slice_in_dim(x, B_TC, B, axis=0).reshape(B_SC * 256, 16)
        ap_sc, pv_sc = _sc_run(x2, act2, mask_i32)
        ap_tc, pv_tc = _tc_run(x, act_row, mask_3d, eye_3d)
        ap = jnp.concatenate([ap_tc, ap_sc.reshape(B_SC, N_ACT)], axis=0)
        pv = jnp.concatenate([pv_tc.reshape(B_TC), pv_sc])
        return ap, pv

    return run(x, act2, mask_i32, act_row, mask_3d, eye_3d)


# speedup vs baseline: 3.3546x; 1.7141x over previous
import functools

import jax
import jax.numpy as jnp
from jax import lax
from jax.experimental import pallas as pl
from jax.experimental.pallas import tpu as pltpu
from jax.experimental.pallas import tpu_sc as plsc

B = 16384
N_OBJ = 64
N_ACT = 8
P_SPACE = 0.1
EXIST_THR = 0.8

B_SC = 1024
B_TC = B - B_SC

NC, NS, L = 2, 16, 16
NW = NC * NS
ROWS_PER_W = B_SC // NW
NB = 16
TILES = ROWS_PER_W // NB
GROW = 256
GAT = NB * GROW


TB = 512
ROW = N_OBJ * N_OBJ


def _tc_body(x_blk, act_v, mask_v, eye_v, ap, pv):
    xb = x_blk[...]
    eqf = ((xb > EXIST_THR) == (mask_v[...] > 0)).astype(jnp.float32)
    s = lax.dot_general(eqf, eye_v[...], (((1,), (0,)), ((), ())),
                        preferred_element_type=jnp.float32)
    eqall = s > (N_OBJ - 0.5)
    a_col = xb[:, 0:1]
    b_col = xb[:, N_OBJ:N_OBJ + 1]
    p = jnp.abs(a_col - b_col)
    satf = jnp.where(eqall & (p < P_SPACE), 1.0, 0.0)
    pv[...] = p
    an = act_v[...] / (act_v[...] + 1e-20)
    ap[...] = satf * an


def _tc_run(x4, act_row, mask_4k, eye_col):
    return pl.pallas_call(
        _tc_body,
        grid=(B_TC // TB,),
        out_shape=[
            jax.ShapeDtypeStruct((B_TC, N_ACT), jnp.float32),
            jax.ShapeDtypeStruct((B_TC, 1), jnp.float32),
        ],
        in_specs=[
            pl.BlockSpec((TB, ROW), lambda t: (t, 0)),
            pl.BlockSpec((1, N_ACT), lambda t: (0, 0)),
            pl.BlockSpec((1, ROW), lambda t: (0, 0)),
            pl.BlockSpec((ROW, 1), lambda t: (0, 0)),
        ],
        out_specs=[
            pl.BlockSpec((TB, N_ACT), lambda t: (t, 0)),
            pl.BlockSpec((TB, 1), lambda t: (t, 0)),
        ],
    )(x4, act_row, mask_4k, eye_col)


def _sc_body(x_hbm, act_hbm, mask_hbm, ap_hbm, pv_hbm,
             gat_v, mask_v, mexp_v, act_v, sat_v, ap_v, pv_v, sem):
    wid = lax.axis_index("s") * NC + lax.axis_index("c")
    base_row = wid * ROWS_PER_W

    pltpu.sync_copy(mask_hbm, mask_v)
    pltpu.sync_copy(act_hbm, act_v)

    iota = lax.iota(jnp.int32, L)
    r256 = iota * GROW

    for m in range(N_OBJ // L):
        chunk = mask_v[pl.ds(m * L, L)]
        for j in range(L):
            mexp_v[pl.ds((m * L + j) * L, L)] = jnp.broadcast_to(
                chunk[j], (L,))

    act = act_v[...]
    an = act / (act + 1e-20)
    half = (iota >= 8).astype(jnp.int32)
    zero = jnp.zeros((L,), jnp.int32)

    def tile(t, carry):
        b0 = base_row + t * NB
        pltpu.sync_copy(x_hbm.at[pl.ds(b0 * GROW, GAT)], gat_v)

        a_val = plsc.load_gather(gat_v, [r256, zero])
        b_val = plsc.load_gather(gat_v, [r256 + 4, zero])
        p = jnp.abs(a_val - b_val)
        acc = p < P_SPACE
        for i in range(N_OBJ):
            rows = r256 + (65 * i) // 16
            col = jnp.full((L,), i % 16, jnp.int32)
            diag = plsc.load_gather(gat_v, [rows, col])
            m_i = mexp_v[pl.ds(i * L, L)] > 0
            acc = acc & ((diag > EXIST_THR) == m_i)
        satf = jnp.where(acc, 1.0, 0.0).astype(jnp.float32)

        pv_v[pl.ds(t * NB, NB)] = p
        sat_v[...] = satf
        for pair in range(NB // 2):
            sel = half + 2 * pair
            ap_v[pl.ds(t * NB * N_ACT + pair * L, L)] = (
                plsc.load_gather(sat_v, [sel]) * an)
        return carry

    lax.fori_loop(0, TILES, tile, 0, unroll=False)

    pltpu.sync_copy(pv_v, pv_hbm.at[pl.ds(wid * ROWS_PER_W, ROWS_PER_W)])
    pltpu.sync_copy(ap_v, ap_hbm.at[pl.ds(wid * ROWS_PER_W * N_ACT,
                                          ROWS_PER_W * N_ACT)])


def _sc_run(x2, act2, mask_i32):
    mesh = plsc.VectorSubcoreMesh(core_axis_name="c", subcore_axis_name="s")
    f = functools.partial(
        pl.kernel,
        mesh=mesh,
        compiler_params=pltpu.CompilerParams(needs_layout_passes=False,
                                             use_tc_tiling_on_sc=False),
        out_type=[
            jax.ShapeDtypeStruct((B_SC * N_ACT,), jnp.float32),
            jax.ShapeDtypeStruct((B_SC,), jnp.float32),
        ],
        scratch_types=[
            pltpu.VMEM((GAT, L), jnp.float32),
            pltpu.VMEM((N_OBJ,), jnp.int32),
            pltpu.VMEM((N_OBJ * L,), jnp.int32),
            pltpu.VMEM((L,), jnp.float32),
            pltpu.VMEM((L,), jnp.float32),
            pltpu.VMEM((ROWS_PER_W * N_ACT,), jnp.float32),
            pltpu.VMEM((ROWS_PER_W,), jnp.float32),
            pltpu.SemaphoreType.DMA,
        ],
    )(_sc_body)
    return f(x2, act2, mask_i32)


def kernel(x, action, mask):
    act2 = jnp.concatenate([action, action]).astype(jnp.float32)
    mask_i32 = mask.astype(jnp.int32)
    act_row = action.astype(jnp.float32).reshape(1, N_ACT)
    mask_4k = jnp.repeat(mask.astype(jnp.int32), N_OBJ).reshape(1, ROW)
    eye_col = jnp.eye(N_OBJ, dtype=jnp.float32).reshape(ROW, 1)

    @jax.jit
    def run(x, act2, mask_i32, act_row, mask_4k, eye_col):
        x2 = lax.slice_in_dim(x, B_TC, B, axis=0).reshape(B_SC * 256, 16)
        ap_sc, pv_sc = _sc_run(x2, act2, mask_i32)
        ap_tc, pv_tc = _tc_run(x.reshape(B, ROW), act_row, mask_4k,
                               eye_col)
        ap = jnp.concatenate([ap_tc, ap_sc.reshape(B_SC, N_ACT)], axis=0)
        pv = jnp.concatenate([pv_tc.reshape(B_TC), pv_sc])
        return ap, pv

    return run(x, act2, mask_i32, act_row, mask_4k, eye_col)
